# baseline (speedup 1.0000x reference)
import functools

import jax
import jax.numpy as jnp
from jax import lax
from jax.experimental import pallas as pl
from jax.experimental.pallas import tpu as pltpu
from jax.experimental.pallas import tpu_sc as plsc

_TROWS = 32


def _make_sc_kernel(B, N, D, dtype):
    R = B * N
    half = N // 2
    wpb = 32 // B
    span = half // wpb
    last_span = span - 8
    mesh = plsc.VectorSubcoreMesh(core_axis_name="c", subcore_axis_name="s")

    @functools.partial(
        pl.kernel,
        mesh=mesh,
        out_type=jax.ShapeDtypeStruct((R, D), dtype),
        scratch_types=[
            pltpu.VMEM((_TROWS, D), dtype),
            pltpu.VMEM((8, D), dtype),
            pltpu.VMEM((8, D), dtype),
            pltpu.SemaphoreType.DMA,
            pltpu.SemaphoreType.DMA,
            pltpu.SemaphoreType.DMA,
            pltpu.SemaphoreType.DMA,
        ],
    )
    def sc_kernel(
        x_hbm, tok_hbm, out_hbm, tbuf, ebuf_a, ebuf_b,
        sem_c, sem_f, sem_e, sem_t,
    ):
        wid = lax.axis_index("s") * 2 + lax.axis_index("c")
        b = wid // wpb
        k = wid % wpb
        is_last = k == wpb - 1
        rb = b * N

        cs = rb + half + k * span
        cp_full = pltpu.make_async_copy(
            x_hbm.at[pl.ds(cs, span)], out_hbm.at[pl.ds(cs, span)], sem_c
        )
        cp_part = pltpu.make_async_copy(
            x_hbm.at[pl.ds(cs, last_span)],
            out_hbm.at[pl.ds(cs, last_span)],
            sem_c,
        )

        @pl.when(jnp.logical_not(is_last))
        def _():
            cp_full.start()

        @pl.when(is_last)
        def _():
            cp_part.start()
            pltpu.make_async_copy(
                x_hbm.at[pl.ds(rb + half - 8, 8)], ebuf_a, sem_e
            ).start()
            pltpu.make_async_copy(
                x_hbm.at[pl.ds(rb + N - 8, 8)], ebuf_b, sem_e
            ).start()

        stages = [
            pltpu.make_async_copy(tok_hbm, tbuf.at[r], sem_t)
            for r in range(_TROWS)
        ]
        for st in stages:
            st.start()
        for st in stages:
            st.wait()

        fs = rb + k * span

        @pl.when(jnp.logical_not(is_last))
        def _():
            fills = [
                pltpu.make_async_copy(
                    tbuf,
                    out_hbm.at[pl.ds(fs + i * _TROWS, _TROWS)],
                    sem_f,
                )
                for i in range(span // _TROWS)
            ]
            for f in fills:
                f.start()
            for f in fills:
                f.wait()
            cp_full.wait()

        @pl.when(is_last)
        def _():
            nf = last_span // _TROWS
            rem = last_span - nf * _TROWS
            fills = [
                pltpu.make_async_copy(
                    tbuf,
                    out_hbm.at[pl.ds(fs + i * _TROWS, _TROWS)],
                    sem_f,
                )
                for i in range(nf)
            ]
            if rem:
                fills.append(
                    pltpu.make_async_copy(
                        tbuf.at[pl.ds(0, rem)],
                        out_hbm.at[pl.ds(fs + nf * _TROWS, rem)],
                        sem_f,
                    )
                )
            for f in fills:
                f.start()
            pltpu.make_async_copy(
                x_hbm.at[pl.ds(rb + half - 8, 8)], ebuf_a, sem_e
            ).wait()
            pltpu.make_async_copy(
                x_hbm.at[pl.ds(rb + N - 8, 8)], ebuf_b, sem_e
            ).wait()
            patches = [
                pltpu.make_async_copy(tok_hbm, ebuf_a.at[r], sem_t)
                for r in range(7)
            ]
            patches.append(
                pltpu.make_async_copy(tok_hbm, ebuf_b.at[7], sem_t)
            )
            for p in patches:
                p.start()
            for p in patches:
                p.wait()
            ea = pltpu.make_async_copy(
                ebuf_a, out_hbm.at[pl.ds(rb + half - 8, 8)], sem_c
            )
            eb = pltpu.make_async_copy(
                ebuf_b, out_hbm.at[pl.ds(rb + N - 8, 8)], sem_c
            )
            ea.start()
            eb.start()
            for f in fills:
                f.wait()
            cp_part.wait()
            ea.wait()
            eb.wait()

    return sc_kernel


def kernel(x, token):
    B, N, D = x.shape
    out = _make_sc_kernel(B, N, D, x.dtype)(x.reshape(B * N, D), token)
    return out.reshape(B, N, D)


# speedup vs baseline: 14.7316x; 14.7316x over previous
import functools

import jax
import jax.numpy as jnp
from jax import lax
from jax.experimental import pallas as pl
from jax.experimental.pallas import tpu as pltpu
from jax.experimental.pallas import tpu_sc as plsc

_TROWS = 32
_CROWS = 16
_NBUF = 4


def _make_sc_kernel(B, N, D, dtype):
    R = B * N
    half = N // 2
    wpb = 32 // B
    span = half // wpb
    last_span = span - 8
    mesh = plsc.VectorSubcoreMesh(core_axis_name="c", subcore_axis_name="s")

    @functools.partial(
        pl.kernel,
        mesh=mesh,
        out_type=jax.ShapeDtypeStruct((R, D), dtype),
        scratch_types=[
            pltpu.VMEM((_TROWS, D), dtype),
            pltpu.VMEM((_NBUF * _CROWS, D), dtype),
            pltpu.VMEM((8, D), dtype),
            pltpu.VMEM((8, D), dtype),
            pltpu.SemaphoreType.DMA,
            pltpu.SemaphoreType.DMA,
            pltpu.SemaphoreType.DMA,
            pltpu.SemaphoreType.DMA,
            pltpu.SemaphoreType.DMA,
        ],
    )
    def sc_kernel(
        x_hbm, tok_hbm, out_hbm, tbuf, cbuf, ebuf_a, ebuf_b,
        sem_ci, sem_co, sem_f, sem_e, sem_t,
    ):
        wid = lax.axis_index("s") * 2 + lax.axis_index("c")
        b = wid // wpb
        k = wid % wpb
        is_last = k == wpb - 1
        rb = b * N

        cs = rb + half + k * span

        def _copy_ring(nrows):
            sizes = [_CROWS] * (nrows // _CROWS)
            if nrows % _CROWS:
                sizes.append(nrows % _CROWS)
            nch = len(sizes)
            offs = [sum(sizes[:i]) for i in range(nch)]
            ins = [
                pltpu.make_async_copy(
                    x_hbm.at[pl.ds(cs + offs[i], sizes[i])],
                    cbuf.at[pl.ds((i % _NBUF) * _CROWS, sizes[i])],
                    sem_ci,
                )
                for i in range(nch)
            ]
            outs = [
                pltpu.make_async_copy(
                    cbuf.at[pl.ds((i % _NBUF) * _CROWS, sizes[i])],
                    out_hbm.at[pl.ds(cs + offs[i], sizes[i])],
                    sem_co,
                )
                for i in range(nch)
            ]
            for i in range(min(_NBUF, nch)):
                ins[i].start()
            for i in range(nch):
                ins[i].wait()
                outs[i].start()
                if i + _NBUF < nch:
                    outs[i].wait()
                    ins[i + _NBUF].start()
            for i in range(max(0, nch - _NBUF), nch):
                outs[i].wait()

        @pl.when(is_last)
        def _():
            pltpu.make_async_copy(
                x_hbm.at[pl.ds(rb + half - 8, 8)], ebuf_a, sem_e
            ).start()
            pltpu.make_async_copy(
                x_hbm.at[pl.ds(rb + N - 8, 8)], ebuf_b, sem_e
            ).start()

        stages = [
            pltpu.make_async_copy(tok_hbm, tbuf.at[r], sem_t)
            for r in range(_TROWS)
        ]
        for st in stages:
            st.start()
        for st in stages:
            st.wait()

        fs = rb + k * span

        @pl.when(jnp.logical_not(is_last))
        def _():
            fills = [
                pltpu.make_async_copy(
                    tbuf,
                    out_hbm.at[pl.ds(fs + i * _TROWS, _TROWS)],
                    sem_f,
                )
                for i in range(span // _TROWS)
            ]
            for f in fills:
                f.start()
            _copy_ring(span)
            for f in fills:
                f.wait()

        @pl.when(is_last)
        def _():
            nf = last_span // _TROWS
            rem = last_span - nf * _TROWS
            fills = [
                pltpu.make_async_copy(
                    tbuf,
                    out_hbm.at[pl.ds(fs + i * _TROWS, _TROWS)],
                    sem_f,
                )
                for i in range(nf)
            ]
            if rem:
                fills.append(
                    pltpu.make_async_copy(
                        tbuf.at[pl.ds(0, rem)],
                        out_hbm.at[pl.ds(fs + nf * _TROWS, rem)],
                        sem_f,
                    )
                )
            for f in fills:
                f.start()
            _copy_ring(last_span)
            pltpu.make_async_copy(
                x_hbm.at[pl.ds(rb + half - 8, 8)], ebuf_a, sem_e
            ).wait()
            pltpu.make_async_copy(
                x_hbm.at[pl.ds(rb + N - 8, 8)], ebuf_b, sem_e
            ).wait()
            patches = [
                pltpu.make_async_copy(tok_hbm, ebuf_a.at[r], sem_t)
                for r in range(7)
            ]
            patches.append(
                pltpu.make_async_copy(tok_hbm, ebuf_b.at[7], sem_t)
            )
            for p in patches:
                p.start()
            for p in patches:
                p.wait()
            ea = pltpu.make_async_copy(
                ebuf_a, out_hbm.at[pl.ds(rb + half - 8, 8)], sem_e
            )
            eb = pltpu.make_async_copy(
                ebuf_b, out_hbm.at[pl.ds(rb + N - 8, 8)], sem_e
            )
            ea.start()
            eb.start()
            for f in fills:
                f.wait()
            ea.wait()
            eb.wait()

    return sc_kernel


def kernel(x, token):
    B, N, D = x.shape
    out = _make_sc_kernel(B, N, D, x.dtype)(x.reshape(B * N, D), token)
    return out.reshape(B, N, D)


# speedup vs baseline: 31.9619x; 2.1696x over previous
import functools

import jax
import jax.numpy as jnp
from jax.experimental import pallas as pl
from jax.experimental.pallas import tpu as pltpu

_TROWS = 2048
_SLOTS = 2


def _body(x_hbm, tok_ref, o_hbm, tbuf, ring, ea, eb,
          sem_f, sem_e, sem_ci, sem_co, *, b_total, n, d):
    half = n // 2
    chunk_sizes = (_TROWS, half - 8 - _TROWS)

    chunks = []
    for b in range(b_total):
        off = half
        for sz in chunk_sizes:
            chunks.append((b, off, sz))
            off += sz
    ins, outs = [], []
    for i, (b, off, sz) in enumerate(chunks):
        s = i % _SLOTS
        ins.append(
            pltpu.make_async_copy(
                x_hbm.at[b, pl.ds(off, sz)],
                ring.at[s, pl.ds(0, sz)],
                sem_ci.at[s],
            )
        )
        outs.append(
            pltpu.make_async_copy(
                ring.at[s, pl.ds(0, sz)],
                o_hbm.at[b, pl.ds(off, sz)],
                sem_co.at[s],
            )
        )

    edge_loads = []
    for b in range(b_total):
        edge_loads.append(
            pltpu.make_async_copy(
                x_hbm.at[b, pl.ds(half - 8, 8)], ea.at[b], sem_e
            )
        )
        edge_loads.append(
            pltpu.make_async_copy(
                x_hbm.at[b, pl.ds(n - 8, 8)], eb.at[b], sem_e
            )
        )

    for cp in edge_loads:
        cp.start()
    for i in range(_SLOTS):
        ins[i].start()

    tbuf[...] = jnp.broadcast_to(tok_ref[...][None, :], (_TROWS, d))

    fills = []
    for b in range(b_total):
        off = 0
        for sz in chunk_sizes:
            fills.append(
                pltpu.make_async_copy(
                    tbuf.at[pl.ds(0, sz)],
                    o_hbm.at[b, pl.ds(off, sz)],
                    sem_f,
                )
            )
            off += sz
    for cp in fills:
        cp.start()

    for i in range(len(chunks)):
        ins[i].wait()
        outs[i].start()
        if i + _SLOTS < len(chunks):
            outs[i].wait()
            ins[i + _SLOTS].start()

    for cp in edge_loads:
        cp.wait()
    ridx = jax.lax.broadcasted_iota(jnp.int32, (b_total, 8, d), 1)
    tok3 = tok_ref[...][None, None, :]
    ea[...] = jnp.where(ridx < 7, tok3, ea[...])
    eb[...] = jnp.where(ridx < 7, eb[...], tok3)
    edge_stores = []
    for b in range(b_total):
        edge_stores.append(
            pltpu.make_async_copy(
                ea.at[b], o_hbm.at[b, pl.ds(half - 8, 8)], sem_e
            )
        )
        edge_stores.append(
            pltpu.make_async_copy(
                eb.at[b], o_hbm.at[b, pl.ds(n - 8, 8)], sem_e
            )
        )
    for cp in edge_stores:
        cp.start()

    for i in range(max(0, len(chunks) - _SLOTS), len(chunks)):
        outs[i].wait()
    for cp in fills:
        cp.wait()
    for cp in edge_stores:
        cp.wait()


def kernel(x, token):
    B, N, D = x.shape

    return pl.pallas_call(
        functools.partial(_body, b_total=B, n=N, d=D),
        in_specs=[
            pl.BlockSpec(memory_space=pl.ANY),
            pl.BlockSpec(memory_space=pltpu.VMEM),
        ],
        out_specs=pl.BlockSpec(memory_space=pl.ANY),
        out_shape=jax.ShapeDtypeStruct((B, N, D), x.dtype),
        scratch_shapes=[
            pltpu.VMEM((_TROWS, D), x.dtype),
            pltpu.VMEM((_SLOTS, _TROWS, D), x.dtype),
            pltpu.VMEM((B, 8, D), x.dtype),
            pltpu.VMEM((B, 8, D), x.dtype),
            pltpu.SemaphoreType.DMA,
            pltpu.SemaphoreType.DMA,
            pltpu.SemaphoreType.DMA((_SLOTS,)),
            pltpu.SemaphoreType.DMA((_SLOTS,)),
        ],
    )(x, token)


# speedup vs baseline: 32.2232x; 1.0082x over previous
import functools

import jax
import jax.numpy as jnp
from jax.experimental import pallas as pl
from jax.experimental.pallas import tpu as pltpu

_TROWS = 1024
_SLOTS = 6


def _body(x_hbm, tok_ref, o_hbm, tbuf, ring, ea, eb,
          sem_f, sem_e, sem_ci, sem_co, *, b_total, n, d):
    half = n // 2
    chunk_sizes = (_TROWS, _TROWS, _TROWS, half - 8 - 3 * _TROWS)

    chunks = []
    for b in range(b_total):
        off = half
        for sz in chunk_sizes:
            chunks.append((b, off, sz))
            off += sz
    ins, outs = [], []
    for i, (b, off, sz) in enumerate(chunks):
        s = i % _SLOTS
        ins.append(
            pltpu.make_async_copy(
                x_hbm.at[b, pl.ds(off, sz)],
                ring.at[s, pl.ds(0, sz)],
                sem_ci.at[s],
            )
        )
        outs.append(
            pltpu.make_async_copy(
                ring.at[s, pl.ds(0, sz)],
                o_hbm.at[b, pl.ds(off, sz)],
                sem_co.at[s],
            )
        )

    edge_loads = []
    for b in range(b_total):
        edge_loads.append(
            pltpu.make_async_copy(
                x_hbm.at[b, pl.ds(half - 8, 8)], ea.at[b], sem_e
            )
        )
        edge_loads.append(
            pltpu.make_async_copy(
                x_hbm.at[b, pl.ds(n - 8, 8)], eb.at[b], sem_e
            )
        )

    for cp in edge_loads:
        cp.start()
    for i in range(_SLOTS):
        ins[i].start()

    tbuf[...] = jnp.broadcast_to(tok_ref[...][None, :], (_TROWS, d))

    fills = []
    for b in range(b_total):
        off = 0
        for sz in chunk_sizes:
            fills.append(
                pltpu.make_async_copy(
                    tbuf.at[pl.ds(0, sz)],
                    o_hbm.at[b, pl.ds(off, sz)],
                    sem_f,
                )
            )
            off += sz
    for cp in fills:
        cp.start()

    nch = len(chunks)
    for i in range(nch):
        if i >= 1 and (i - 1) + _SLOTS < nch:
            outs[i - 1].wait()
            ins[i - 1 + _SLOTS].start()
        ins[i].wait()
        outs[i].start()

    for cp in edge_loads:
        cp.wait()
    ridx = jax.lax.broadcasted_iota(jnp.int32, (b_total, 8, d), 1)
    tok3 = tok_ref[...][None, None, :]
    ea[...] = jnp.where(ridx < 7, tok3, ea[...])
    eb[...] = jnp.where(ridx < 7, eb[...], tok3)
    edge_stores = []
    for b in range(b_total):
        edge_stores.append(
            pltpu.make_async_copy(
                ea.at[b], o_hbm.at[b, pl.ds(half - 8, 8)], sem_e
            )
        )
        edge_stores.append(
            pltpu.make_async_copy(
                eb.at[b], o_hbm.at[b, pl.ds(n - 8, 8)], sem_e
            )
        )
    for cp in edge_stores:
        cp.start()

    for i in range(max(0, len(chunks) - _SLOTS), len(chunks)):
        outs[i].wait()
    for cp in fills:
        cp.wait()
    for cp in edge_stores:
        cp.wait()


def kernel(x, token):
    B, N, D = x.shape

    return pl.pallas_call(
        functools.partial(_body, b_total=B, n=N, d=D),
        in_specs=[
            pl.BlockSpec(memory_space=pl.ANY),
            pl.BlockSpec(memory_space=pltpu.VMEM),
        ],
        out_specs=pl.BlockSpec(memory_space=pl.ANY),
        out_shape=jax.ShapeDtypeStruct((B, N, D), x.dtype),
        scratch_shapes=[
            pltpu.VMEM((_TROWS, D), x.dtype),
            pltpu.VMEM((_SLOTS, _TROWS, D), x.dtype),
            pltpu.VMEM((B, 8, D), x.dtype),
            pltpu.VMEM((B, 8, D), x.dtype),
            pltpu.SemaphoreType.DMA,
            pltpu.SemaphoreType.DMA,
            pltpu.SemaphoreType.DMA((_SLOTS,)),
            pltpu.SemaphoreType.DMA((_SLOTS,)),
        ],
    )(x, token)


# speedup vs baseline: 32.4449x; 1.0069x over previous
import functools

import jax
import jax.numpy as jnp
from jax.experimental import pallas as pl


def _body(x_ref, edge_ref, tok_ref, o_ref, *, blk, half, n):
    j = pl.program_id(1)
    nblk = half // blk
    tok = tok_ref[...][None, None, None, :]

    @pl.when(j != nblk - 1)
    def _():
        o_ref[:, 0:1] = jnp.broadcast_to(tok, (1, 1, blk, o_ref.shape[3]))
        o_ref[:, 1:2] = x_ref[...]

    @pl.when(j == nblk - 1)
    def _():
        rows = j * blk + jax.lax.broadcasted_iota(
            jnp.int32, (1, 1, blk, 1), 2
        )
        reg0 = jnp.where(
            rows == half - 1, edge_ref[...][:, :, None, :], tok
        )
        reg1 = jnp.where(rows + half <= n - 2, x_ref[...], tok)
        o_ref[:, 0:1] = reg0
        o_ref[:, 1:2] = reg1


def kernel(x, token):
    B, N, D = x.shape
    half = N // 2
    BLK = 2048
    nblk = half // BLK

    x4 = x.reshape(B, 2, half, D)
    edge = jax.lax.slice_in_dim(x, half - 1, half, axis=1).reshape(B, 1, D)

    out = pl.pallas_call(
        functools.partial(_body, blk=BLK, half=half, n=N),
        grid=(B, nblk),
        in_specs=[
            pl.BlockSpec((1, 1, BLK, D), lambda b, j: (b, 1, j, 0)),
            pl.BlockSpec((1, 1, D), lambda b, j: (b, 0, 0)),
            pl.BlockSpec((D,), lambda b, j: (0,)),
        ],
        out_specs=pl.BlockSpec((1, 2, BLK, D), lambda b, j: (b, 0, j, 0)),
        out_shape=jax.ShapeDtypeStruct((B, 2, half, D), x.dtype),
    )(x4, edge, token)
    return out.reshape(B, N, D)
